# baseline (speedup 1.0000x reference)
import functools

import jax
import jax.numpy as jnp
from jax import lax
from jax.experimental import pallas as pl
from jax.experimental.pallas import tpu as pltpu
from jax.experimental.pallas import tpu_sc as plsc

BATCH = 4
SEQ = 8192
DIM = 768

NUM_CORES = 2
NUM_SUBCORES = 16
NW = NUM_CORES * NUM_SUBCORES
ROWS = BATCH * SEQ
RPW = ROWS // NW
CH = 64
CHW = CH * DIM
NCH = RPW // CH
LANES = 16


@functools.partial(
    pl.kernel,
    mesh=plsc.VectorSubcoreMesh(core_axis_name="c", subcore_axis_name="s"),
    out_type=jax.ShapeDtypeStruct((ROWS * DIM,), jnp.float32),
    scratch_types=[
        pltpu.VMEM((CHW,), jnp.float32),
        pltpu.VMEM((CHW,), jnp.float32),
    ],
)
def _posadd(x_hbm, w_hbm, out_hbm, xbuf, wbuf):
    wid = lax.axis_index("s") * NUM_CORES + lax.axis_index("c")
    xbase = wid * (RPW * DIM)
    wbase = lax.rem(wid, NW // BATCH) * (RPW * DIM)

    for c in range(NCH):
        goff = xbase + c * CHW
        woff = wbase + c * CHW
        pltpu.sync_copy(x_hbm.at[pl.ds(goff, CHW)], xbuf)
        pltpu.sync_copy(w_hbm.at[pl.ds(woff, CHW)], wbuf)

        def body(i, carry):
            off = i * LANES
            xbuf[pl.ds(off, LANES)] = (
                xbuf[pl.ds(off, LANES)] + wbuf[pl.ds(off, LANES)]
            )
            return carry

        lax.fori_loop(0, CHW // LANES, body, 0)
        pltpu.sync_copy(xbuf, out_hbm.at[pl.ds(goff, CHW)])


def kernel(x, pos_embed_weight):
    out = _posadd(x.reshape(-1), pos_embed_weight.reshape(-1))
    return out.reshape(x.shape)


# speedup vs baseline: 1.7880x; 1.7880x over previous
import functools

import jax
import jax.numpy as jnp
from jax import lax
from jax.experimental import pallas as pl
from jax.experimental.pallas import tpu as pltpu
from jax.experimental.pallas import tpu_sc as plsc

BATCH = 4
SEQ = 8192
DIM = 768

NUM_CORES = 2
NUM_SUBCORES = 16
NW = NUM_CORES * NUM_SUBCORES
SPW = SEQ // NW
CH = 32
CHW = CH * DIM
NSC = SPW // CH
NCHUNK = NSC * BATCH
LANES = 16
UNROLL = 8


@functools.partial(
    pl.kernel,
    mesh=plsc.VectorSubcoreMesh(core_axis_name="c", subcore_axis_name="s"),
    out_type=jax.ShapeDtypeStruct((BATCH * SEQ * DIM,), jnp.float32),
    scratch_types=[
        pltpu.VMEM((CHW,), jnp.float32),
        pltpu.VMEM((CHW,), jnp.float32),
        pltpu.VMEM((CHW,), jnp.float32),
        pltpu.SemaphoreType.DMA,
        pltpu.SemaphoreType.DMA,
        pltpu.SemaphoreType.DMA,
        pltpu.SemaphoreType.DMA,
    ],
)
def _posadd(x_hbm, w_hbm, out_hbm, wbuf, xbuf0, xbuf1, is0, is1, os0, os1):
    wid = lax.axis_index("s") * NUM_CORES + lax.axis_index("c")
    wrow0 = wid * SPW

    xbufs = (xbuf0, xbuf1)
    in_sems = (is0, is1)
    out_sems = (os0, os1)

    def x_off(k):
        sc, b = divmod(k, BATCH)
        return (b * SEQ + wrow0 + sc * CH) * DIM

    def start_in(k):
        return pltpu.async_copy(
            x_hbm.at[pl.ds(x_off(k), CHW)], xbufs[k % 2], in_sems[k % 2]
        )

    in_descs = [None] * NCHUNK
    out_descs = [None] * NCHUNK
    in_descs[0] = start_in(0)

    for k in range(NCHUNK):
        buf = xbufs[k % 2]
        sc, b = divmod(k, BATCH)
        if b == 0:
            pltpu.sync_copy(w_hbm.at[pl.ds((wrow0 + sc * CH) * DIM, CHW)], wbuf)
        in_descs[k].wait()
        if k >= 2:
            out_descs[k - 2].wait()
        if k + 1 < NCHUNK:
            in_descs[k + 1] = start_in(k + 1)

        @plsc.parallel_loop(0, CHW, LANES, unroll=UNROLL)
        def _(i):
            plsc.addupdate(buf.at[pl.ds(i, LANES)], wbuf[pl.ds(i, LANES)])

        out_descs[k] = pltpu.async_copy(
            buf, out_hbm.at[pl.ds(x_off(k), CHW)], out_sems[k % 2]
        )

    out_descs[NCHUNK - 2].wait()
    out_descs[NCHUNK - 1].wait()


def kernel(x, pos_embed_weight):
    out = _posadd(x.reshape(-1), pos_embed_weight.reshape(-1))
    return out.reshape(x.shape)
